# baseline (speedup 1.0000x reference)
import jax
import jax.numpy as jnp
from jax.experimental import pallas as pl
from jax.experimental.pallas import tpu as pltpu

_C_BLK = 128


def _prelu_body(route_ref, w_ref, f_ref, x_ref, o_ref):
    slope = (w_ref[0, 0] + f_ref[0])[:, None]
    xv = x_ref[...]
    o_ref[...] = jnp.where(xv >= 0, xv, slope * xv)


def kernel(input, route_index, weight, weight_fact):
    B, C, H, W = input.shape
    HW = H * W
    x = input.reshape(B * C, HW)
    routes = route_index.astype(jnp.int32)
    w3 = weight.reshape(weight.shape[0], 1, C)
    n_c = C // _C_BLK

    grid = (B * n_c,)
    grid_spec = pltpu.PrefetchScalarGridSpec(
        num_scalar_prefetch=1,
        grid=grid,
        in_specs=[
            pl.BlockSpec((1, 1, _C_BLK), lambda j, r: (r[j // n_c], 0, j % n_c)),
            pl.BlockSpec((1, _C_BLK), lambda j, r: (0, j % n_c)),
            pl.BlockSpec((_C_BLK, HW), lambda j, r: (j, 0)),
        ],
        out_specs=pl.BlockSpec((_C_BLK, HW), lambda j, r: (j, 0)),
    )
    out = pl.pallas_call(
        _prelu_body,
        grid_spec=grid_spec,
        out_shape=jax.ShapeDtypeStruct((B * C, HW), jnp.float32),
        compiler_params=pltpu.CompilerParams(
            dimension_semantics=("arbitrary",),
        ),
    )(routes, w3, weight_fact, x)
    return out.reshape(B, C, H, W)


# speedup vs baseline: 8.5569x; 8.5569x over previous
import jax
import jax.numpy as jnp
from jax.experimental import pallas as pl
from jax.experimental.pallas import tpu as pltpu

_ROW_BLK = 2048


def _prelu_body(route_ref, w_ref, f_ref, x_ref, o_ref):
    slope = (w_ref[0, 0] + f_ref[0])[None, :]
    xv = x_ref[0]
    o_ref[0] = jnp.where(xv >= 0, xv, slope * xv)


def kernel(input, route_index, weight, weight_fact):
    B, C, H, W = input.shape
    HW = H * W
    routes = route_index.astype(jnp.int32)
    w3 = weight.reshape(weight.shape[0], 1, C)
    x3 = input.transpose(0, 2, 3, 1).reshape(B, HW, C)
    n_j = HW // _ROW_BLK

    grid_spec = pltpu.PrefetchScalarGridSpec(
        num_scalar_prefetch=1,
        grid=(B, n_j),
        in_specs=[
            pl.BlockSpec((1, 1, C), lambda b, j, r: (r[b], 0, 0)),
            pl.BlockSpec((1, C), lambda b, j, r: (0, 0)),
            pl.BlockSpec((1, _ROW_BLK, C), lambda b, j, r: (b, j, 0)),
        ],
        out_specs=pl.BlockSpec((1, _ROW_BLK, C), lambda b, j, r: (b, j, 0)),
    )
    out = pl.pallas_call(
        _prelu_body,
        grid_spec=grid_spec,
        out_shape=jax.ShapeDtypeStruct((B, HW, C), jnp.float32),
        compiler_params=pltpu.CompilerParams(
            dimension_semantics=("arbitrary", "arbitrary"),
        ),
    )(routes, w3, weight_fact, x3)
    return out.reshape(B, H, W, C).transpose(0, 3, 1, 2)


# speedup vs baseline: 8.8423x; 1.0333x over previous
import jax
import jax.numpy as jnp
from jax.experimental import pallas as pl
from jax.experimental.pallas import tpu as pltpu

_ROW_BLK = 4096


def _prelu_body(route_ref, w_ref, f_ref, x_ref, o_ref):
    slope = (w_ref[0, 0] + f_ref[0])[None, :]
    xv = x_ref[0]
    o_ref[0] = jnp.where(xv >= 0, xv, slope * xv)


def kernel(input, route_index, weight, weight_fact):
    B, C, H, W = input.shape
    HW = H * W
    routes = route_index.astype(jnp.int32)
    w3 = weight.reshape(weight.shape[0], 1, C)
    x3 = input.transpose(0, 2, 3, 1).reshape(B, HW, C)
    n_j = HW // _ROW_BLK

    grid_spec = pltpu.PrefetchScalarGridSpec(
        num_scalar_prefetch=1,
        grid=(B, n_j),
        in_specs=[
            pl.BlockSpec((1, 1, C), lambda b, j, r: (r[b], 0, 0)),
            pl.BlockSpec((1, C), lambda b, j, r: (0, 0)),
            pl.BlockSpec((1, _ROW_BLK, C), lambda b, j, r: (b, j, 0)),
        ],
        out_specs=pl.BlockSpec((1, _ROW_BLK, C), lambda b, j, r: (b, j, 0)),
    )
    out = pl.pallas_call(
        _prelu_body,
        grid_spec=grid_spec,
        out_shape=jax.ShapeDtypeStruct((B, HW, C), jnp.float32),
        compiler_params=pltpu.CompilerParams(
            dimension_semantics=("arbitrary", "arbitrary"),
        ),
    )(routes, w3, weight_fact, x3)
    return out.reshape(B, H, W, C).transpose(0, 3, 1, 2)


# speedup vs baseline: 8.9473x; 1.0119x over previous
import jax
import jax.numpy as jnp
from jax.experimental import pallas as pl
from jax.experimental.pallas import tpu as pltpu


def _prelu_body(route_ref, w_ref, f_ref, x_ref, o_ref):
    b = pl.program_id(0)
    idx = route_ref[b]
    slope = (w_ref[idx] + f_ref[0])[None, :]
    xv = x_ref[0]
    o_ref[0] = jnp.where(xv >= 0, xv, slope * xv)


def kernel(input, route_index, weight, weight_fact):
    B, C, H, W = input.shape
    HW = H * W
    routes = route_index.astype(jnp.int32)
    x3 = input.transpose(0, 2, 3, 1).reshape(B, HW, C)

    grid_spec = pltpu.PrefetchScalarGridSpec(
        num_scalar_prefetch=1,
        grid=(B,),
        in_specs=[
            pl.BlockSpec(memory_space=pltpu.VMEM),
            pl.BlockSpec(memory_space=pltpu.VMEM),
            pl.BlockSpec((1, HW, C), lambda b, r: (b, 0, 0)),
        ],
        out_specs=pl.BlockSpec((1, HW, C), lambda b, r: (b, 0, 0)),
    )
    out = pl.pallas_call(
        _prelu_body,
        grid_spec=grid_spec,
        out_shape=jax.ShapeDtypeStruct((B, HW, C), jnp.float32),
        compiler_params=pltpu.CompilerParams(
            dimension_semantics=("arbitrary",),
        ),
    )(routes, weight, weight_fact, x3)
    return out.reshape(B, H, W, C).transpose(0, 3, 1, 2)
